# baseline (speedup 1.0000x reference)
import functools

import jax
import jax.numpy as jnp
from jax.experimental import pallas as pl
from jax.experimental.pallas import tpu as pltpu


def _se_kernel(x_ref, w1_ref, b1_ref, w2_ref, b2_ref, o_ref, *, inv_hw, hw):
    pooled = jnp.sum(x_ref[...], axis=-1, dtype=jnp.float32) * inv_hw
    h = jnp.maximum(
        jax.lax.dot_general(pooled, w1_ref[...],
                            (((1,), (1,)), ((), ())),
                            preferred_element_type=jnp.float32)
        + b1_ref[...], 0.0)
    g = jax.nn.sigmoid(
        jax.lax.dot_general(h, w2_ref[...],
                            (((1,), (1,)), ((), ())),
                            preferred_element_type=jnp.float32)
        + b2_ref[...])
    o_ref[...] = (x_ref[:, :, :hw] * g[:, :, None]).astype(o_ref.dtype)


def _pick_images_per_block(n, bytes_per_image, budget):
    best = 1
    for d in range(1, n + 1):
        if n % d == 0 and d * bytes_per_image <= budget:
            best = d
    return best


def kernel(x_nchw, w1, b1, w2, b2):
    N, C, H, W = x_nchw.shape
    Cr = w1.shape[0]
    HW = H * W
    dtype = x_nchw.dtype

    x3 = x_nchw.reshape(N, C, HW)
    HWp = ((HW + 127) // 128) * 128
    xp = jnp.pad(x3, ((0, 0), (0, 0), (0, HWp - HW))) if HWp != HW else x3
    b1r = b1.reshape(1, Cr)
    b2r = b2.reshape(1, C)
    inv_hw = 1.0 / float(HW)

    bytes_per_image = C * HWp * dtype.itemsize
    nb = _pick_images_per_block(N, bytes_per_image, budget=8 << 20)
    grid = (N // nb,)

    out3 = pl.pallas_call(
        functools.partial(_se_kernel, inv_hw=inv_hw, hw=HW),
        out_shape=jax.ShapeDtypeStruct((N, C, HW), dtype),
        grid=grid,
        in_specs=[
            pl.BlockSpec((nb, C, HWp), lambda i: (i, 0, 0)),
            pl.BlockSpec((Cr, C), lambda i: (0, 0)),
            pl.BlockSpec((1, Cr), lambda i: (0, 0)),
            pl.BlockSpec((C, Cr), lambda i: (0, 0)),
            pl.BlockSpec((1, C), lambda i: (0, 0)),
        ],
        out_specs=pl.BlockSpec((nb, C, HW), lambda i: (i, 0, 0)),
        compiler_params=pltpu.CompilerParams(
            dimension_semantics=("arbitrary",),
            allow_input_fusion=[True, False, False, False, False],
            vmem_limit_bytes=48 << 20,
        ),
    )(xp, w1, b1r, w2, b2r)

    return out3.reshape(N, C, H, W)


# speedup vs baseline: 1.2673x; 1.2673x over previous
import functools

import jax
import jax.numpy as jnp
from jax.experimental import pallas as pl
from jax.experimental.pallas import tpu as pltpu


def _se_kernel(x_ref, w1_ref, b1_ref, w2_ref, b2_ref, o_ref, *, inv_hw, hw):
    x = x_ref[...]
    pooled = jnp.sum(x[:, :, :hw], axis=-1, dtype=jnp.float32) * inv_hw
    h = jnp.maximum(
        jax.lax.dot_general(pooled, w1_ref[...],
                            (((1,), (1,)), ((), ())),
                            preferred_element_type=jnp.float32)
        + b1_ref[...], 0.0)
    g = jax.nn.sigmoid(
        jax.lax.dot_general(h, w2_ref[...],
                            (((1,), (1,)), ((), ())),
                            preferred_element_type=jnp.float32)
        + b2_ref[...])
    o_ref[...] = (x * g[:, :, None]).astype(o_ref.dtype)


def _pick_images_per_block(n, bytes_per_image, budget):
    best = 1
    for d in range(1, n + 1):
        if n % d == 0 and d * bytes_per_image <= budget:
            best = d
    return best


def kernel(x_nchw, w1, b1, w2, b2):
    N, C, H, W = x_nchw.shape
    Cr = w1.shape[0]
    HW = H * W
    dtype = x_nchw.dtype

    x3 = x_nchw.reshape(N, C, HW)
    HWp = ((HW + 127) // 128) * 128
    b1r = b1.reshape(1, Cr)
    b2r = b2.reshape(1, C)
    inv_hw = 1.0 / float(HW)

    bytes_per_image = C * HWp * dtype.itemsize
    nb = _pick_images_per_block(N, bytes_per_image, budget=8 << 20)
    grid = (N // nb,)

    out3 = pl.pallas_call(
        functools.partial(_se_kernel, inv_hw=inv_hw, hw=HW),
        out_shape=jax.ShapeDtypeStruct((N, C, HW), dtype),
        grid=grid,
        in_specs=[
            pl.BlockSpec((nb, C, HWp), lambda i: (i, 0, 0)),
            pl.BlockSpec((Cr, C), lambda i: (0, 0)),
            pl.BlockSpec((1, Cr), lambda i: (0, 0)),
            pl.BlockSpec((C, Cr), lambda i: (0, 0)),
            pl.BlockSpec((1, C), lambda i: (0, 0)),
        ],
        out_specs=pl.BlockSpec((nb, C, HWp), lambda i: (i, 0, 0)),
        compiler_params=pltpu.CompilerParams(
            dimension_semantics=("arbitrary",),
            vmem_limit_bytes=48 << 20,
            disable_bounds_checks=True,
        ),
    )(x3, w1, b1r, w2, b2r)

    return out3.reshape(N, C, H, W)
